# baseline (speedup 1.0000x reference)
import functools

import jax
import jax.numpy as jnp
from jax.experimental import pallas as pl
from jax.experimental.pallas import tpu as pltpu

NS = 16
IN_SZ = 256
HID_SZ = 512
OUT_SZ = 256
N = IN_SZ + HID_SZ + OUT_SZ

CHUNK_A = 32
CHUNK_B = 32


def _stage_a_body(obs_ref, ps_ref, adj_ref, h_ref, scal_ref,
                  wfc_ref, bfc_ref, wih_ref, whh_ref, bih_ref, bhh_ref,
                  s_ref):
    c = pl.program_id(0)
    s = pl.program_id(1)
    rew = scal_ref[0]
    lr = scal_ref[1]

    Wfc = wfc_ref[...]
    rs = jnp.sum(Wfc, axis=1)
    obs_c = obs_ref[0, 0]
    pre = jnp.tanh(obs_c[:, None] * rs[None, :] + bfc_ref[...][None, :])

    Wih = wih_ref[...]
    A = (pre @ Wih[:, :NS].T + (rew * Wih[:, 2 * NS])[None, :]
         + bih_ref[...][None, :])
    Bf = ps_ref[...] @ Wih[:, NS:2 * NS].T

    h = h_ref[...]
    R, C = h.shape[0], h.shape[1]
    h2 = h.reshape(R * C, NS)
    gh = h2 @ whh_ref[...].T + bhh_ref[...][None, :]
    gi = (A[:, None, :] + Bf[None, :, :]).reshape(R * C, 3 * NS)
    ir, iz, i_n = gi[:, :NS], gi[:, NS:2 * NS], gi[:, 2 * NS:]
    hr, hz, h_n = gh[:, :NS], gh[:, NS:2 * NS], gh[:, 2 * NS:]
    r = jax.nn.sigmoid(ir + hr)
    z = jax.nn.sigmoid(iz + hz)
    n = jnp.tanh(i_n + r * h_n)
    upd = h2 + ((1.0 - z) * n + z * h2) * lr
    syn = upd.reshape(R, C, NS) * pre[:, None, :]
    contrib = jnp.sum(adj_ref[...][:, :, None] * syn, axis=0)

    @pl.when(s == 0)
    def _():
        s_ref[...] = jnp.zeros_like(s_ref)

    s_ref[...] += contrib


def _stage_b_body(shid_ref, ps_ref, adj_ref, h_ref, scal_ref,
                  wfc_ref, bfc_ref, wfco_ref, bfco_ref,
                  wih_ref, whh_ref, bih_ref, bhh_ref,
                  x_ref, pre_ref, sout_ref):
    s = pl.program_id(0)
    nsteps = pl.num_programs(0)
    rew = scal_ref[0]
    lr = scal_ref[1]

    @pl.when(s == 0)
    def _():
        post_hid = jnp.tanh(shid_ref[...] @ wfc_ref[...].T
                            + bfc_ref[...][None, :])
        pre_ref[...] = post_hid
        sout_ref[...] = jnp.zeros_like(sout_ref)

    pre = pre_ref[pl.ds(s * CHUNK_B, CHUNK_B), :]
    Wih = wih_ref[...]
    A = (pre @ Wih[:, :NS].T + (rew * Wih[:, 2 * NS])[None, :]
         + bih_ref[...][None, :])
    Bf = ps_ref[...] @ Wih[:, NS:2 * NS].T

    h = h_ref[...]
    R, C = h.shape[0], h.shape[1]
    h2 = h.reshape(R * C, NS)
    gh = h2 @ whh_ref[...].T + bhh_ref[...][None, :]
    gi = (A[:, None, :] + Bf[None, :, :]).reshape(R * C, 3 * NS)
    ir, iz, i_n = gi[:, :NS], gi[:, NS:2 * NS], gi[:, 2 * NS:]
    hr, hz, h_n = gh[:, :NS], gh[:, NS:2 * NS], gh[:, 2 * NS:]
    r = jax.nn.sigmoid(ir + hr)
    z = jax.nn.sigmoid(iz + hz)
    n = jnp.tanh(i_n + r * h_n)
    upd = h2 + ((1.0 - z) * n + z * h2) * lr
    syn = upd.reshape(R, C, NS) * pre[:, None, :]
    sout_ref[...] += jnp.sum(adj_ref[...][:, :, None] * syn, axis=0)

    @pl.when(s == nsteps - 1)
    def _():
        w0 = wfco_ref[...][0, :]
        x = jax.lax.dot_general(w0[None, :], sout_ref[...],
                                (((1,), (1,)), ((), ())))
        x_ref[...] = x + bfco_ref[0]


def _run(obs, reward, adjacency, hidden_state, post_state, lr,
         W_fc_in, b_fc_in, W_fc_hid, b_fc_hid, W_fc_out, b_fc_out,
         Wih_in, Whh_in, bih_in, bhh_in,
         Wih_hid, Whh_hid, bih_hid, bhh_hid, interpret=False):
    scal = jnp.stack([reward[0], lr]).astype(jnp.float32)
    obs2 = obs.reshape(IN_SZ // CHUNK_A, 1, CHUNK_A)

    grid_a = (2, IN_SZ // CHUNK_A)
    s_hid = pl.pallas_call(
        _stage_a_body,
        grid=grid_a,
        in_specs=[
            pl.BlockSpec((1, 1, CHUNK_A), lambda c, s: (s, 0, 0)),
            pl.BlockSpec((256, NS), lambda c, s: (1 + c, 0)),
            pl.BlockSpec((CHUNK_A, 256), lambda c, s: (s, 1 + c)),
            pl.BlockSpec((CHUNK_A, 256, NS), lambda c, s: (s, 1 + c, 0)),
            pl.BlockSpec(memory_space=pltpu.SMEM),
            pl.BlockSpec((NS, NS), lambda c, s: (0, 0)),
            pl.BlockSpec((NS,), lambda c, s: (0,)),
            pl.BlockSpec((3 * NS, 2 * NS + 1), lambda c, s: (0, 0)),
            pl.BlockSpec((3 * NS, NS), lambda c, s: (0, 0)),
            pl.BlockSpec((3 * NS,), lambda c, s: (0,)),
            pl.BlockSpec((3 * NS,), lambda c, s: (0,)),
        ],
        out_specs=pl.BlockSpec((256, NS), lambda c, s: (c, 0)),
        out_shape=jax.ShapeDtypeStruct((HID_SZ, NS), jnp.float32),
        interpret=interpret,
    )(obs2, post_state, adjacency, hidden_state, scal,
      W_fc_in, b_fc_in, Wih_in, Whh_in, bih_in, bhh_in)

    grid_b = (HID_SZ // CHUNK_B,)
    x = pl.pallas_call(
        _stage_b_body,
        grid=grid_b,
        in_specs=[
            pl.BlockSpec((HID_SZ, NS), lambda s: (0, 0)),
            pl.BlockSpec((256, NS), lambda s: (3, 0)),
            pl.BlockSpec((CHUNK_B, 256), lambda s: (8 + s, 3)),
            pl.BlockSpec((CHUNK_B, 256, NS), lambda s: (8 + s, 3, 0)),
            pl.BlockSpec(memory_space=pltpu.SMEM),
            pl.BlockSpec((NS, NS), lambda s: (0, 0)),
            pl.BlockSpec((NS,), lambda s: (0,)),
            pl.BlockSpec((NS, NS), lambda s: (0, 0)),
            pl.BlockSpec((NS,), lambda s: (0,)),
            pl.BlockSpec((3 * NS, 2 * NS + 1), lambda s: (0, 0)),
            pl.BlockSpec((3 * NS, NS), lambda s: (0, 0)),
            pl.BlockSpec((3 * NS,), lambda s: (0,)),
            pl.BlockSpec((3 * NS,), lambda s: (0,)),
        ],
        out_specs=pl.BlockSpec((1, OUT_SZ), lambda s: (0, 0)),
        out_shape=jax.ShapeDtypeStruct((1, OUT_SZ), jnp.float32),
        scratch_shapes=[pltpu.VMEM((HID_SZ, NS), jnp.float32),
                        pltpu.VMEM((OUT_SZ, NS), jnp.float32)],
        interpret=interpret,
    )(s_hid, post_state, adjacency, hidden_state, scal,
      W_fc_hid, b_fc_hid, W_fc_out, b_fc_out,
      Wih_hid, Whh_hid, bih_hid, bhh_hid)

    scores = jnp.tanh(x[0])
    return jnp.argmax(scores)


def kernel(obs, reward, adjacency, hidden_state, post_state, lr,
           W_fc_in, b_fc_in, W_fc_hid, b_fc_hid, W_fc_out, b_fc_out,
           Wih_in, Whh_in, bih_in, bhh_in,
           Wih_hid, Whh_hid, bih_hid, bhh_hid,
           Wih_out, Whh_out, bih_out, bhh_out):
    del Wih_out, Whh_out, bih_out, bhh_out
    return _run(obs, reward, adjacency, hidden_state, post_state, lr,
                W_fc_in, b_fc_in, W_fc_hid, b_fc_hid, W_fc_out, b_fc_out,
                Wih_in, Whh_in, bih_in, bhh_in,
                Wih_hid, Whh_hid, bih_hid, bhh_hid)


# speedup vs baseline: 1.5047x; 1.5047x over previous
import jax
import jax.numpy as jnp
from jax.experimental import pallas as pl
from jax.experimental.pallas import tpu as pltpu

NS = 16
IN_SZ = 256
HID_SZ = 512
OUT_SZ = 256
N = IN_SZ + HID_SZ + OUT_SZ

CHUNK = 32

_DN = (((1,), (1,)), ((), ()))


def _gru_cols(k, ghT, hT, BT, AchT, preT, adj_ref, lr, acc):
    C = BT.shape[1]
    sl = slice(k * C, (k + 1) * C)
    gh_k = ghT[:, sl]
    h_k = hT[:, sl]
    gi_k = BT + AchT[:, k:k + 1]
    r = jax.nn.sigmoid(gi_k[:NS] + gh_k[:NS])
    z = jax.nn.sigmoid(gi_k[NS:2 * NS] + gh_k[NS:2 * NS])
    n = jnp.tanh(gi_k[2 * NS:] + r * gh_k[2 * NS:])
    upd = h_k + ((1.0 - z) * n + z * h_k) * lr
    syn = upd * preT[:, k:k + 1]
    return acc + adj_ref[k:k + 1, :] * syn


def _stage_a_body(obs_ref, ps_ref, adj_ref, h_ref, scal_ref,
                  wfc_ref, bfc_ref, wih_ref, whh_ref, bih_ref, bhh_ref,
                  s_ref):
    s = pl.program_id(1)
    rew = scal_ref[0]
    lr = scal_ref[1]

    rs = jnp.sum(wfc_ref[...], axis=1)
    obs_row = obs_ref[0, 0]
    preT = jnp.tanh(rs[:, None] * obs_row[None, :] + bfc_ref[...][:, None])

    Wih = wih_ref[...]
    cvec = rew * Wih[:, 2 * NS] + bih_ref[...]
    AchT = Wih[:, :NS] @ preT + cvec[:, None]
    BT = jax.lax.dot_general(Wih[:, NS:2 * NS], ps_ref[...], _DN)

    h2 = h_ref[...].reshape(CHUNK * 256, NS)
    ghT = (jax.lax.dot_general(whh_ref[...], h2, _DN)
           + bhh_ref[...][:, None])
    hT = jax.lax.dot_general(jnp.eye(NS, dtype=jnp.float32), h2, _DN)

    acc = jnp.zeros((NS, 256), jnp.float32)
    for k in range(CHUNK):
        acc = _gru_cols(k, ghT, hT, BT, AchT, preT, adj_ref, lr, acc)

    @pl.when(s == 0)
    def _():
        s_ref[...] = jnp.zeros_like(s_ref)

    s_ref[...] += acc


def _stage_b_body(shid_ref, ps_ref, adj_ref, h_ref, scal_ref,
                  wfc_ref, bfc_ref, wfco_ref, bfco_ref,
                  wih_ref, whh_ref, bih_ref, bhh_ref,
                  x_ref, pre_ref, a_ref, sout_ref):
    s = pl.program_id(0)
    nsteps = pl.num_programs(0)
    rew = scal_ref[0]
    lr = scal_ref[1]
    Wih = wih_ref[...]

    @pl.when(s == 0)
    def _():
        preT = jnp.tanh(wfc_ref[...] @ shid_ref[...]
                        + bfc_ref[...][:, None])
        pre_ref[...] = preT
        cvec = rew * Wih[:, 2 * NS] + bih_ref[...]
        a_ref[...] = Wih[:, :NS] @ preT + cvec[:, None]
        sout_ref[...] = jnp.zeros_like(sout_ref)

    iota = jax.lax.broadcasted_iota(jnp.int32, (HID_SZ, CHUNK), 0)
    kidx = jax.lax.broadcasted_iota(jnp.int32, (HID_SZ, CHUNK), 1)
    OH = jnp.where(iota == s * CHUNK + kidx, 1.0, 0.0)
    preT = pre_ref[...] @ OH
    AchT = a_ref[...] @ OH

    BT = jax.lax.dot_general(Wih[:, NS:2 * NS], ps_ref[...], _DN)
    h2 = h_ref[...].reshape(CHUNK * 256, NS)
    ghT = (jax.lax.dot_general(whh_ref[...], h2, _DN)
           + bhh_ref[...][:, None])
    hT = jax.lax.dot_general(jnp.eye(NS, dtype=jnp.float32), h2, _DN)

    acc = jnp.zeros((NS, 256), jnp.float32)
    for k in range(CHUNK):
        acc = _gru_cols(k, ghT, hT, BT, AchT, preT, adj_ref, lr, acc)
    sout_ref[...] += acc

    @pl.when(s == nsteps - 1)
    def _():
        x_ref[...] = wfco_ref[0:1, :] @ sout_ref[...] + bfco_ref[0]


def _run(obs, reward, adjacency, hidden_state, post_state, lr,
         W_fc_in, b_fc_in, W_fc_hid, b_fc_hid, W_fc_out, b_fc_out,
         Wih_in, Whh_in, bih_in, bhh_in,
         Wih_hid, Whh_hid, bih_hid, bhh_hid, interpret=False):
    scal = jnp.stack([reward[0], lr]).astype(jnp.float32)
    obs2 = obs.reshape(IN_SZ // CHUNK, 1, CHUNK)

    grid_a = (2, IN_SZ // CHUNK)
    s_hid_t = pl.pallas_call(
        _stage_a_body,
        grid=grid_a,
        in_specs=[
            pl.BlockSpec((1, 1, CHUNK), lambda c, s: (s, 0, 0)),
            pl.BlockSpec((256, NS), lambda c, s: (1 + c, 0)),
            pl.BlockSpec((CHUNK, 256), lambda c, s: (s, 1 + c)),
            pl.BlockSpec((CHUNK, 256, NS), lambda c, s: (s, 1 + c, 0)),
            pl.BlockSpec(memory_space=pltpu.SMEM),
            pl.BlockSpec((NS, NS), lambda c, s: (0, 0)),
            pl.BlockSpec((NS,), lambda c, s: (0,)),
            pl.BlockSpec((3 * NS, 2 * NS + 1), lambda c, s: (0, 0)),
            pl.BlockSpec((3 * NS, NS), lambda c, s: (0, 0)),
            pl.BlockSpec((3 * NS,), lambda c, s: (0,)),
            pl.BlockSpec((3 * NS,), lambda c, s: (0,)),
        ],
        out_specs=pl.BlockSpec((NS, 256), lambda c, s: (0, c)),
        out_shape=jax.ShapeDtypeStruct((NS, HID_SZ), jnp.float32),
        interpret=interpret,
    )(obs2, post_state, adjacency, hidden_state, scal,
      W_fc_in, b_fc_in, Wih_in, Whh_in, bih_in, bhh_in)

    grid_b = (HID_SZ // CHUNK,)
    x = pl.pallas_call(
        _stage_b_body,
        grid=grid_b,
        in_specs=[
            pl.BlockSpec((NS, HID_SZ), lambda s: (0, 0)),
            pl.BlockSpec((256, NS), lambda s: (3, 0)),
            pl.BlockSpec((CHUNK, 256), lambda s: (8 + s, 3)),
            pl.BlockSpec((CHUNK, 256, NS), lambda s: (8 + s, 3, 0)),
            pl.BlockSpec(memory_space=pltpu.SMEM),
            pl.BlockSpec((NS, NS), lambda s: (0, 0)),
            pl.BlockSpec((NS,), lambda s: (0,)),
            pl.BlockSpec((NS, NS), lambda s: (0, 0)),
            pl.BlockSpec((NS,), lambda s: (0,)),
            pl.BlockSpec((3 * NS, 2 * NS + 1), lambda s: (0, 0)),
            pl.BlockSpec((3 * NS, NS), lambda s: (0, 0)),
            pl.BlockSpec((3 * NS,), lambda s: (0,)),
            pl.BlockSpec((3 * NS,), lambda s: (0,)),
        ],
        out_specs=pl.BlockSpec((1, OUT_SZ), lambda s: (0, 0)),
        out_shape=jax.ShapeDtypeStruct((1, OUT_SZ), jnp.float32),
        scratch_shapes=[pltpu.VMEM((NS, HID_SZ), jnp.float32),
                        pltpu.VMEM((3 * NS, HID_SZ), jnp.float32),
                        pltpu.VMEM((NS, OUT_SZ), jnp.float32)],
        interpret=interpret,
    )(s_hid_t, post_state, adjacency, hidden_state, scal,
      W_fc_hid, b_fc_hid, W_fc_out, b_fc_out,
      Wih_hid, Whh_hid, bih_hid, bhh_hid)

    scores = jnp.tanh(x[0])
    return jnp.argmax(scores)


def kernel(obs, reward, adjacency, hidden_state, post_state, lr,
           W_fc_in, b_fc_in, W_fc_hid, b_fc_hid, W_fc_out, b_fc_out,
           Wih_in, Whh_in, bih_in, bhh_in,
           Wih_hid, Whh_hid, bih_hid, bhh_hid,
           Wih_out, Whh_out, bih_out, bhh_out):
    del Wih_out, Whh_out, bih_out, bhh_out
    return _run(obs, reward, adjacency, hidden_state, post_state, lr,
                W_fc_in, b_fc_in, W_fc_hid, b_fc_hid, W_fc_out, b_fc_out,
                Wih_in, Whh_in, bih_in, bhh_in,
                Wih_hid, Whh_hid, bih_hid, bhh_hid)
